# Initial kernel scaffold; baseline (speedup 1.0000x reference)
import jax
import jax.numpy as jnp
from jax.experimental import pallas as pl


def kernel(inputs, table):
    raise NotImplementedError("write your pallas kernel here")


# speedup vs baseline: 1.4938x; 1.4938x over previous
import functools

import jax
import jax.numpy as jnp
from jax import lax
from jax.experimental import pallas as pl
from jax.experimental.pallas import tpu as pltpu
from jax.experimental.pallas import tpu_sc as plsc

_D = 32
_NC = 2
_NS = 16
_NW = _NC * _NS


@functools.partial(jax.jit, static_argnames=("chunk",))
def _sc_embedding_gather(table, idx_flat, *, chunk):
    b = idx_flat.shape[0]
    b_per_w = b // _NW
    n_chunks = b_per_w // chunk
    mesh = plsc.VectorSubcoreMesh(core_axis_name="c", subcore_axis_name="s")

    @functools.partial(
        pl.kernel,
        out_type=jax.ShapeDtypeStruct((b, _D), jnp.float32),
        mesh=mesh,
        scratch_types=[
            pltpu.VMEM((chunk,), jnp.int32),
            pltpu.VMEM((chunk, _D), jnp.float32),
            pltpu.SemaphoreType.DMA,
        ],
        compiler_params=pltpu.CompilerParams(use_tc_tiling_on_sc=False),
    )
    def k(table_hbm, idx_hbm, out_hbm, idx_v, rows_v, sem):
        wid = lax.axis_index("s") * _NC + lax.axis_index("c")
        base = wid * b_per_w

        def body(i, carry):
            off = base + i * chunk
            pltpu.sync_copy(idx_hbm.at[pl.ds(off, chunk)], idx_v)
            pltpu.async_copy(table_hbm.at[idx_v], rows_v, sem).wait()
            pltpu.sync_copy(rows_v, out_hbm.at[pl.ds(off, chunk)])
            return carry

        lax.fori_loop(0, n_chunks, body, 0)

    return k(table, idx_flat)


def kernel(inputs, table):
    b, s = inputs.shape
    idx_flat = inputs.reshape(b * s).astype(jnp.int32)
    out = _sc_embedding_gather(table, idx_flat, chunk=3200)
    return out.reshape(b, s, _D)


# speedup vs baseline: 1.5015x; 1.0051x over previous
import functools

import jax
import jax.numpy as jnp
from jax import lax
from jax.experimental import pallas as pl
from jax.experimental.pallas import tpu as pltpu
from jax.experimental.pallas import tpu_sc as plsc

_D = 32
_NC = 2
_NS = 16
_NW = _NC * _NS
_NBUF = 4


@functools.partial(jax.jit, static_argnames=("chunk",))
def _sc_embedding_gather(table, idx_flat, *, chunk):
    b = idx_flat.shape[0]
    b_per_w = b // _NW
    n_chunks = b_per_w // chunk
    assert n_chunks % _NBUF == 0 and n_chunks >= _NBUF
    mesh = plsc.VectorSubcoreMesh(core_axis_name="c", subcore_axis_name="s")

    @functools.partial(
        pl.kernel,
        out_type=jax.ShapeDtypeStruct((b, _D), jnp.float32),
        mesh=mesh,
        scratch_types=(
            [pltpu.VMEM((chunk,), jnp.int32) for _ in range(_NBUF)]
            + [pltpu.VMEM((chunk, _D), jnp.float32) for _ in range(_NBUF)]
            + [pltpu.SemaphoreType.DMA for _ in range(2 * _NBUF)]
        ),
        compiler_params=pltpu.CompilerParams(use_tc_tiling_on_sc=False),
    )
    def k(table_hbm, idx_hbm, out_hbm, *scratch):
        idx_v = scratch[:_NBUF]
        rows_v = scratch[_NBUF : 2 * _NBUF]
        gsem = scratch[2 * _NBUF : 3 * _NBUF]
        ssem = scratch[3 * _NBUF : 4 * _NBUF]
        wid = lax.axis_index("s") * _NC + lax.axis_index("c")
        base = wid * b_per_w

        def start_gather(ci, slot):
            off = base + ci * chunk
            pltpu.sync_copy(idx_hbm.at[pl.ds(off, chunk)], idx_v[slot])
            pltpu.async_copy(
                table_hbm.at[idx_v[slot]], rows_v[slot], gsem[slot]
            )

        def wait_gather(slot):
            pltpu.make_async_copy(
                table_hbm.at[idx_v[slot]], rows_v[slot], gsem[slot]
            ).wait()

        def start_store(ci, slot):
            off = base + ci * chunk
            pltpu.async_copy(
                rows_v[slot], out_hbm.at[pl.ds(off, chunk)], ssem[slot]
            )

        def wait_store(ci, slot):
            off = base + ci * chunk
            pltpu.make_async_copy(
                rows_v[slot], out_hbm.at[pl.ds(off, chunk)], ssem[slot]
            ).wait()

        start_gather(0, 0)
        start_gather(1, 1)

        def body(p, carry):
            for s in range(_NBUF):
                i = p * _NBUF + s
                wait_gather(s)
                start_store(i, s)
                nxt = (s + 2) % _NBUF

                @pl.when(jnp.logical_and(i + 2 < n_chunks, i >= 2))
                def _():
                    wait_store(i - 2, nxt)

                @pl.when(i + 2 < n_chunks)
                def _():
                    start_gather(i + 2, nxt)
            return carry

        lax.fori_loop(0, n_chunks // _NBUF, body, 0)
        for j in range(n_chunks - _NBUF, n_chunks):
            wait_store(j, j % _NBUF)

    return k(table, idx_flat)


def kernel(inputs, table):
    b, s = inputs.shape
    idx_flat = inputs.reshape(b * s).astype(jnp.int32)
    out = _sc_embedding_gather(table, idx_flat, chunk=800)
    return out.reshape(b, s, _D)
